# baseline (speedup 1.0000x reference)
import jax
import jax.numpy as jnp
from jax.experimental import pallas as pl
from jax.experimental.pallas import tpu as pltpu


def _copy_kernel(perm_ref, in_ref, out_ref):
    out_ref[...] = in_ref[...]


def kernel(sources):
    s2, bs, c, t = sources.shape
    perm_key = jax.random.key(42)
    perm = jnp.argsort(jax.random.uniform(perm_key, (bs,))).astype(jnp.int32)

    grid = (s2, bs)

    lanes = 128
    subl = (c * t) // lanes
    src = sources.reshape(s2, bs, subl, lanes)

    def in_index(s, b, perm_ref):
        row = jnp.where(s == 0, perm_ref[b], b)
        return (s, row, 0, 0)

    def out_index(s, b, perm_ref):
        return (s, b, 0, 0)

    out = pl.pallas_call(
        _copy_kernel,
        grid_spec=pltpu.PrefetchScalarGridSpec(
            num_scalar_prefetch=1,
            grid=grid,
            in_specs=[pl.BlockSpec((1, 1, subl, lanes), in_index)],
            out_specs=pl.BlockSpec((1, 1, subl, lanes), out_index),
        ),
        out_shape=jax.ShapeDtypeStruct(src.shape, src.dtype),
    )(perm, src)
    return out.reshape(sources.shape)
